# baseline (speedup 1.0000x reference)
import functools
import math

import jax
import jax.numpy as jnp
from jax import lax
from jax.experimental import pallas as pl
from jax.experimental.pallas import tpu as pltpu
from jax.experimental.pallas import tpu_sc as plsc

D = 64
SCALE = math.sqrt(D)
LANES = 16

NC = 2
NS = 16
NW = NC * NS

IDX_ROW = 128
K = 8
CHUNK = K * IDX_ROW


def _make_lookup(b_total: int):
    assert b_total % (NW * CHUNK) == 0
    b_per_w = b_total // NW
    n_chunks = b_per_w // CHUNK
    idx_rows_per_w = b_per_w // IDX_ROW

    mesh = plsc.VectorSubcoreMesh(core_axis_name="c", subcore_axis_name="s")

    @functools.partial(
        pl.kernel,
        out_type=jax.ShapeDtypeStruct((b_total, D), jnp.float32),
        mesh=mesh,
        scratch_types=[
            pltpu.VMEM((K, IDX_ROW), jnp.int32),
            pltpu.VMEM((CHUNK, D), jnp.float32),
            pltpu.SemaphoreType.DMA,
        ],
        compiler_params=pltpu.CompilerParams(use_tc_tiling_on_sc=False),
    )
    def lookup(x_hbm, w_hbm, out_hbm, idx_v, rows_v, sem):
        wid = lax.axis_index("s") * NC + lax.axis_index("c")
        row0 = wid * idx_rows_per_w
        out0 = wid * b_per_w

        def chunk_body(g, _):
            pltpu.sync_copy(x_hbm.at[pl.ds(row0 + g * K, K)], idx_v)
            copies = []
            for j in range(K):
                copies.append(
                    pltpu.async_copy(
                        w_hbm.at[idx_v.at[j]],
                        rows_v.at[pl.ds(j * IDX_ROW, IDX_ROW)],
                        sem,
                    )
                )
            for c in copies:
                c.wait()

            def scale_row(r, _):
                for cc in range(D // LANES):
                    sl = pl.ds(cc * LANES, LANES)
                    rows_v[r, sl] = rows_v[r, sl] * SCALE
                return ()

            lax.fori_loop(0, CHUNK, scale_row, ())
            pltpu.sync_copy(rows_v, out_hbm.at[pl.ds(out0 + g * CHUNK, CHUNK)])
            return ()

        lax.fori_loop(0, n_chunks, chunk_body, ())

    return lookup


def kernel(x, W):
    b, t = x.shape
    b_total = b * t
    x2d = x.reshape(b_total // IDX_ROW, IDX_ROW).astype(jnp.int32)
    out = _make_lookup(b_total)(x2d, W)
    return out.reshape(b, t, D)


# speedup vs baseline: 1.1065x; 1.1065x over previous
import functools
import math

import jax
import jax.numpy as jnp
from jax import lax
from jax.experimental import pallas as pl
from jax.experimental.pallas import tpu as pltpu
from jax.experimental.pallas import tpu_sc as plsc

D = 64
SCALE = math.sqrt(D)
LANES = 16

NC = 2
NS = 16
NW = NC * NS

IDX_ROW = 128
K = 2
CHUNK = K * IDX_ROW
NBUF = 4


def _make_lookup(b_total: int):
    assert b_total % (NW * CHUNK * NBUF) == 0
    b_per_w = b_total // NW
    n_chunks = b_per_w // CHUNK
    n_groups = n_chunks // NBUF
    idx_rows_per_w = b_per_w // IDX_ROW

    mesh = plsc.VectorSubcoreMesh(core_axis_name="c", subcore_axis_name="s")

    @functools.partial(
        pl.kernel,
        out_type=jax.ShapeDtypeStruct((b_total, D), jnp.float32),
        mesh=mesh,
        scratch_types=[
            pltpu.VMEM((idx_rows_per_w, IDX_ROW), jnp.int32),
            pltpu.VMEM((NBUF * CHUNK, D), jnp.float32),
            pltpu.SemaphoreType.DMA((NBUF,)),
            pltpu.SemaphoreType.DMA((NBUF,)),
        ],
        compiler_params=pltpu.CompilerParams(use_tc_tiling_on_sc=False),
    )
    def lookup(x_hbm, w_hbm, out_hbm, idx_v, rows_v, gsem, ssem):
        wid = lax.axis_index("s") * NC + lax.axis_index("c")
        out0 = wid * b_per_w

        pltpu.sync_copy(x_hbm.at[pl.ds(wid * idx_rows_per_w, idx_rows_per_w)],
                        idx_v)

        def fire_gather(buf, c):
            for j in range(K):
                pltpu.async_copy(
                    w_hbm.at[idx_v.at[K * c + j]],
                    rows_v.at[pl.ds(buf * CHUNK + j * IDX_ROW, IDX_ROW)],
                    gsem.at[buf],
                )

        def wait_gather(buf):
            pltpu.make_async_copy(
                w_hbm.at[pl.ds(0, CHUNK)],
                rows_v.at[pl.ds(buf * CHUNK, CHUNK)],
                gsem.at[buf],
            ).wait()

        def fire_store(buf, c):
            pltpu.async_copy(
                rows_v.at[pl.ds(buf * CHUNK, CHUNK)],
                out_hbm.at[pl.ds(out0 + c * CHUNK, CHUNK)],
                ssem.at[buf],
            )

        def wait_store(buf):
            pltpu.make_async_copy(
                rows_v.at[pl.ds(buf * CHUNK, CHUNK)],
                out_hbm.at[pl.ds(out0, CHUNK)],
                ssem.at[buf],
            ).wait()

        def scale(buf):
            @plsc.parallel_loop(buf * CHUNK, (buf + 1) * CHUNK, step=1,
                                unroll=8)
            def _(r):
                for cc in range(D // LANES):
                    sl = pl.ds(cc * LANES, LANES)
                    rows_v[r, sl] = rows_v[r, sl] * SCALE

        def step(b, c, first, last_group):
            wait_gather(b)
            scale(b)
            fire_store(b, c)
            prev = (b - 1) % NBUF
            if not first:
                wait_store(prev)
            c2 = c + NBUF - 1
            if not last_group or c2 < n_chunks:
                fire_gather(prev, c2)

        for b in range(NBUF - 1):
            fire_gather(b, b)

        for b in range(NBUF):
            step(b, b, first=(b == 0), last_group=False)

        def group(m, _):
            for b in range(NBUF):
                step(b, m * NBUF + b, first=False, last_group=False)
            return ()

        lax.fori_loop(1, n_groups - 1, group, ())

        for b in range(NBUF):
            step(b, (n_groups - 1) * NBUF + b, first=False, last_group=True)

        wait_store(NBUF - 1)

    return lookup


def kernel(x, W):
    b, t = x.shape
    b_total = b * t
    x2d = x.reshape(b_total // IDX_ROW, IDX_ROW).astype(jnp.int32)
    out = _make_lookup(b_total)(x2d, W)
    return out.reshape(b, t, D)
